# baseline (speedup 1.0000x reference)
import dataclasses

import jax
import jax.numpy as jnp
from jax import lax
from jax.experimental import pallas as pl
from jax.experimental.pallas import tpu as pltpu
from jax.experimental.pallas import tpu_sc as plsc

B = 2
Q = 2048
N = 16384
K_NN = 32
L = 16
NCHUNK = N // L
NGROUP = NCHUNK // 16
NSUPER = NGROUP // 16
QPT = Q // 16
NQ = 4

_BIG = 1 << 20
_INF = float("inf")


def _bf16_round(x):
  u = lax.bitcast_convert_type(x, jnp.int32)
  rounded = u + 0x7FFF + (lax.shift_right_logical(u, 16) & 1)
  masked = rounded & jnp.int32(-65536)
  return lax.bitcast_convert_type(masked, jnp.float32)


def _sqrt16(x):
  i = lax.bitcast_convert_type(x, jnp.int32)
  i = 0x5F3759DF - lax.shift_right_logical(i, 1)
  y = lax.bitcast_convert_type(i, jnp.float32)
  for _ in range(4):
    y = y * (1.5 - 0.5 * x * y * y)
  return x * y


def _tree_min(vals):
  while len(vals) > 1:
    vals = [jnp.minimum(a, b) for a, b in zip(vals[::2], vals[1::2])] + (
        [vals[-1]] if len(vals) % 2 else [])
  return vals[0]


def _sc_body(src_hbm, tgt_hbm, outd_hbm, outi_hbm,
             t2_ref, r2_ref, q_ref,
             d0_ref, d1_ref, d2_ref, d3_ref,
             gmin0_ref, gmin1_ref, gmin2_ref, gmin3_ref,
             smin0_ref, smin1_ref, smin2_ref, smin3_ref,
             xf0_ref, xf1_ref, xf2_ref, xf3_ref,
             xi0_ref, xi1_ref, xi2_ref, xi3_ref,
             od_ref, oi_ref, sem):
  xfl = (xf0_ref, xf1_ref, xf2_ref, xf3_ref)
  xil = (xi0_ref, xi1_ref, xi2_ref, xi3_ref)
  dl = (d0_ref, d1_ref, d2_ref, d3_ref)
  gl = (gmin0_ref, gmin1_ref, gmin2_ref, gmin3_ref)
  sml = (smin0_ref, smin1_ref, smin2_ref, smin3_ref)
  c = lax.axis_index("core")
  s = lax.axis_index("subcore")

  for comp in range(3):
    pltpu.async_copy(tgt_hbm.at[c, comp], dl[comp], sem).wait()
  pltpu.async_copy(src_hbm.at[c, :, pl.ds(s * QPT, QPT)], q_ref, sem).wait()

  @pl.loop(0, NCHUNK)
  def _(j):
    sl16 = pl.ds(j * L, L)
    tx = dl[0][sl16]
    ty = dl[1][sl16]
    tz = dl[2][sl16]
    r2_ref[sl16] = tx * tx + ty * ty + tz * tz
    txb = lax.bitcast_convert_type(_bf16_round(tx), jnp.int32)
    tyb = lax.bitcast_convert_type(_bf16_round(ty), jnp.int32)
    word = txb | lax.shift_right_logical(tyb, 16)
    t2_ref[0, sl16] = lax.bitcast_convert_type(word, jnp.float32)
    t2_ref[1, sl16] = _bf16_round(tz)

  lanes = lax.iota(jnp.int32, L)
  rots = [(lanes + sh) & 15 for sh in (8, 4, 2, 1)]

  def _xmin(v, tmp):
    for r in rots:
      tmp[...] = v
      v = jnp.minimum(v, plsc.load_gather(tmp, [r]))
    return v

  def _xsum(v, tmp):
    for r in rots:
      tmp[...] = v
      v = v + plsc.load_gather(tmp, [r])
    return v

  def _lane_splat(vec, off, tmp):
    return _xmin(jnp.where(lanes == off, vec, _INF), tmp)

  def _first_match(rows, m, tmp):
    return _xmin(_tree_min(
        [jnp.where(row == m, t, _BIG) for t, row in enumerate(rows)]), tmp)[0]

  @pl.loop(0, QPT, step=NQ)
  def _(qi):
    qs = []
    for p in range(NQ):
      qq = qi + p
      b16 = qq & (-16)
      off = qq - b16
      qx = _lane_splat(q_ref[0, pl.ds(b16, L)], off, xfl[p])
      qy = _lane_splat(q_ref[1, pl.ds(b16, L)], off, xfl[p])
      qz = _lane_splat(q_ref[2, pl.ds(b16, L)], off, xfl[p])
      q2 = qx * qx + qy * qy + qz * qz
      qs.append((_bf16_round(qx), _bf16_round(qy), _bf16_round(qz), q2))

    @pl.loop(0, NGROUP)
    def _(g):
      gacc = [jnp.full((L,), _INF, jnp.float32) for _ in range(NQ)]
      for t in range(16):
        sl16 = pl.ds((g * 16 + t) * L, L)
        w = lax.bitcast_convert_type(t2_ref[0, sl16], jnp.int32)
        tx = lax.bitcast_convert_type(w & jnp.int32(-65536), jnp.float32)
        ty = lax.bitcast_convert_type(lax.shift_left(w, 16), jnp.float32)
        tz = t2_ref[1, sl16]
        r2 = r2_ref[sl16]
        for p in range(NQ):
          qx, qy, qz, q2 = qs[p]
          dot = tx * qx + ty * qy + tz * qz
          key = (q2 + r2) - 2.0 * dot
          dl[p][sl16] = key
          gacc[p] = jnp.minimum(gacc[p], key)
      for p in range(NQ):
        gl[p][g] = gacc[p]

    for p in range(NQ):
      for ss in range(NSUPER):
        sml[p][ss] = _tree_min([gl[p][ss * 16 + t] for t in range(16)])

    def round_body(k, carry):
      new_carry = []
      for p in range(NQ):
        d0, d1, i0, i1 = carry[p]
        tt = jnp.minimum(jnp.minimum(sml[p][0], sml[p][1]),
                         jnp.minimum(sml[p][2], sml[p][3]))
        m = _xmin(tt, xfl[p])
        s_star = _first_match([sml[p][ss] for ss in range(NSUPER)], m, xil[p])
        g_star = s_star * 16 + _first_match(
            [gl[p][s_star * 16 + t] for t in range(16)], m, xil[p])
        j_rel = _first_match(
            [dl[p][pl.ds((g_star * 16 + t) * L, L)] for t in range(16)], m,
            xil[p])
        c_star = g_star * 16 + j_rel
        row = dl[p][pl.ds(c_star * L, L)]
        l_star = _xmin(jnp.where(row == m, lanes, _BIG), xil[p])
        idx = c_star * L + l_star
        dl[p][pl.ds(c_star * L, L)] = jnp.where(lanes == l_star, _INF, row)
        gl[p][g_star] = _tree_min(
            [dl[p][pl.ds((g_star * 16 + t) * L, L)] for t in range(16)])
        sml[p][s_star] = _tree_min(
            [gl[p][s_star * 16 + t] for t in range(16)])
        d0 = jnp.where(lanes == k, m, d0)
        d1 = jnp.where(lanes == k - 16, m, d1)
        i0 = jnp.where(lanes == k, idx, i0)
        i1 = jnp.where(lanes == k - 16, idx, i1)
        new_carry.append((d0, d1, i0, i1))
      return tuple(new_carry)

    init = tuple(
        (jnp.zeros((L,), jnp.float32), jnp.zeros((L,), jnp.float32),
         jnp.zeros((L,), jnp.int32), jnp.zeros((L,), jnp.int32))
        for _ in range(NQ))
    res = lax.fori_loop(0, K_NN, round_body, init)

    for p in range(NQ):
      d0, d1, i0, i1 = res[p]
      v0 = _sqrt16(jnp.maximum(d0, 1e-12))
      v1 = _sqrt16(jnp.maximum(d1, 1e-12))
      tot = _xsum(v0 + v1, xfl[p])
      od_ref[qi + p, pl.ds(0, L)] = v0 / tot
      od_ref[qi + p, pl.ds(L, L)] = v1 / tot
      oi_ref[qi + p, pl.ds(0, L)] = i0
      oi_ref[qi + p, pl.ds(L, L)] = i1

  pltpu.async_copy(od_ref, outd_hbm.at[c, s], sem).wait()
  pltpu.async_copy(oi_ref, outi_hbm.at[c, s], sem).wait()


@jax.jit
def kernel(src_pts, tgt_pts):
  src_xyz = src_pts[:, :3, :]
  tgt_xyz = tgt_pts[:, :3, :]

  mesh = plsc.VectorSubcoreMesh(core_axis_name="core", subcore_axis_name="subcore")
  cp = pltpu.CompilerParams(use_tc_tiling_on_sc=False)
  if "needs_layout_passes" in pltpu.CompilerParams.__dataclass_fields__:
    cp = dataclasses.replace(cp, needs_layout_passes=False)

  fn = pl.kernel(
      _sc_body,
      out_type=(
          jax.ShapeDtypeStruct((B, 16, QPT, K_NN), jnp.float32),
          jax.ShapeDtypeStruct((B, 16, QPT, K_NN), jnp.int32),
      ),
      mesh=mesh,
      scratch_types=[
          pltpu.VMEM((2, N), jnp.float32),
          pltpu.VMEM((N,), jnp.float32),
          pltpu.VMEM((3, QPT), jnp.float32),
          pltpu.VMEM((N,), jnp.float32),
          pltpu.VMEM((N,), jnp.float32),
          pltpu.VMEM((N,), jnp.float32),
          pltpu.VMEM((N,), jnp.float32),
          pltpu.VMEM((NGROUP, L), jnp.float32),
          pltpu.VMEM((NGROUP, L), jnp.float32),
          pltpu.VMEM((NGROUP, L), jnp.float32),
          pltpu.VMEM((NGROUP, L), jnp.float32),
          pltpu.VMEM((NSUPER, L), jnp.float32),
          pltpu.VMEM((NSUPER, L), jnp.float32),
          pltpu.VMEM((NSUPER, L), jnp.float32),
          pltpu.VMEM((NSUPER, L), jnp.float32),
          pltpu.VMEM((L,), jnp.float32),
          pltpu.VMEM((L,), jnp.float32),
          pltpu.VMEM((L,), jnp.float32),
          pltpu.VMEM((L,), jnp.float32),
          pltpu.VMEM((L,), jnp.int32),
          pltpu.VMEM((L,), jnp.int32),
          pltpu.VMEM((L,), jnp.int32),
          pltpu.VMEM((L,), jnp.int32),
          pltpu.VMEM((QPT, K_NN), jnp.float32),
          pltpu.VMEM((QPT, K_NN), jnp.int32),
          pltpu.SemaphoreType.DMA,
      ],
      compiler_params=cp,
  )
  outd, outi = fn(src_xyz, tgt_xyz)
  return outd.reshape(B, Q, K_NN), outi.reshape(B, Q, K_NN)


# speedup vs baseline: 1.3919x; 1.3919x over previous
import dataclasses

import jax
import jax.numpy as jnp
from jax import lax
from jax.experimental import pallas as pl
from jax.experimental.pallas import tpu as pltpu
from jax.experimental.pallas import tpu_sc as plsc

B = 2
Q = 2048
N = 16384
K_NN = 32
L = 16
NCHUNK = N // L
NGROUP = NCHUNK // 16
NSUPER = NGROUP // 16
QPT = Q // 16
NQ = 4

_BIG = 1 << 20
_INF = float("inf")


def _bf16_round(x):
  u = lax.bitcast_convert_type(x, jnp.int32)
  rounded = u + 0x7FFF + (lax.shift_right_logical(u, 16) & 1)
  masked = rounded & jnp.int32(-65536)
  return lax.bitcast_convert_type(masked, jnp.float32)


def _sqrt16(x):
  i = lax.bitcast_convert_type(x, jnp.int32)
  i = 0x5F3759DF - lax.shift_right_logical(i, 1)
  y = lax.bitcast_convert_type(i, jnp.float32)
  for _ in range(4):
    y = y * (1.5 - 0.5 * x * y * y)
  return x * y


def _tree_min(vals):
  while len(vals) > 1:
    vals = [jnp.minimum(a, b) for a, b in zip(vals[::2], vals[1::2])] + (
        [vals[-1]] if len(vals) % 2 else [])
  return vals[0]


def _sc_body(src_hbm, tgt_hbm, outd_hbm, outi_hbm,
             t2_ref, r2_ref, q_ref,
             d0_ref, d1_ref, d2_ref, d3_ref,
             gmin0_ref, gmin1_ref, gmin2_ref, gmin3_ref,
             smin0_ref, smin1_ref, smin2_ref, smin3_ref,
             od_ref, oi_ref, sem):
  dl = (d0_ref, d1_ref, d2_ref, d3_ref)
  gl = (gmin0_ref, gmin1_ref, gmin2_ref, gmin3_ref)
  sml = (smin0_ref, smin1_ref, smin2_ref, smin3_ref)
  c = lax.axis_index("core")
  s = lax.axis_index("subcore")

  for comp in range(3):
    pltpu.async_copy(tgt_hbm.at[c, comp], dl[comp], sem).wait()
  pltpu.async_copy(src_hbm.at[c, :, pl.ds(s * QPT, QPT)], q_ref, sem).wait()

  @pl.loop(0, NCHUNK)
  def _(j):
    sl16 = pl.ds(j * L, L)
    tx = dl[0][sl16]
    ty = dl[1][sl16]
    tz = dl[2][sl16]
    r2_ref[sl16] = tx * tx + ty * ty + tz * tz
    txb = lax.bitcast_convert_type(_bf16_round(tx), jnp.int32)
    tyb = lax.bitcast_convert_type(_bf16_round(ty), jnp.int32)
    word = txb | lax.shift_right_logical(tyb, 16)
    t2_ref[0, sl16] = lax.bitcast_convert_type(word, jnp.float32)
    t2_ref[1, sl16] = _bf16_round(tz)

  lanes = lax.iota(jnp.int32, L)

  def _lane_scalar(vec, off):
    return jnp.min(jnp.where(lanes == off, vec, _INF))

  def _first_match(rows, m):
    return jnp.min(_tree_min(
        [jnp.where(row == m, t, _BIG) for t, row in enumerate(rows)]))

  @pl.loop(0, QPT, step=NQ)
  def _(qi):
    qs = []
    for p in range(NQ):
      qq = qi + p
      b16 = qq & (-16)
      off = qq - b16
      qx = _lane_scalar(q_ref[0, pl.ds(b16, L)], off)
      qy = _lane_scalar(q_ref[1, pl.ds(b16, L)], off)
      qz = _lane_scalar(q_ref[2, pl.ds(b16, L)], off)
      q2 = qx * qx + qy * qy + qz * qz
      qs.append((_bf16_round(qx), _bf16_round(qy), _bf16_round(qz), q2))

    @pl.loop(0, NGROUP)
    def _(g):
      gacc = [jnp.full((L,), _INF, jnp.float32) for _ in range(NQ)]
      for t in range(16):
        sl16 = pl.ds((g * 16 + t) * L, L)
        w = lax.bitcast_convert_type(t2_ref[0, sl16], jnp.int32)
        tx = lax.bitcast_convert_type(w & jnp.int32(-65536), jnp.float32)
        ty = lax.bitcast_convert_type(lax.shift_left(w, 16), jnp.float32)
        tz = t2_ref[1, sl16]
        r2 = r2_ref[sl16]
        for p in range(NQ):
          qx, qy, qz, q2 = qs[p]
          dot = tx * qx + ty * qy + tz * qz
          key = (q2 + r2) - 2.0 * dot
          dl[p][sl16] = key
          gacc[p] = jnp.minimum(gacc[p], key)
      for p in range(NQ):
        gl[p][g] = gacc[p]

    for p in range(NQ):
      for ss in range(NSUPER):
        sml[p][ss] = _tree_min([gl[p][ss * 16 + t] for t in range(16)])

    def round_body(k, carry):
      new_carry = []
      for p in range(NQ):
        d0, d1, i0, i1 = carry[p]
        srows = [sml[p][ss] for ss in range(NSUPER)]
        m = jnp.min(_tree_min(list(srows)))
        s_star = _first_match(srows, m)
        grows = [gl[p][s_star * 16 + t] for t in range(16)]
        g_rel = _first_match(grows, m)
        g_star = s_star * 16 + g_rel
        drows = [dl[p][pl.ds((g_star * 16 + t) * L, L)] for t in range(16)]
        j_rel = _first_match(drows, m)
        c_star = g_star * 16 + j_rel
        row = dl[p][pl.ds(c_star * L, L)]
        l_star = jnp.min(jnp.where(row == m, lanes, _BIG))
        idx = c_star * L + l_star
        newrow = jnp.where(lanes == l_star, _INF, row)
        dl[p][pl.ds(c_star * L, L)] = newrow
        newg = _tree_min(
            [jnp.where(j_rel == t, newrow, r) for t, r in enumerate(drows)])
        gl[p][g_star] = newg
        news = _tree_min(
            [jnp.where(g_rel == t, newg, r) for t, r in enumerate(grows)])
        sml[p][s_star] = news
        d0 = jnp.where(lanes == k, m, d0)
        d1 = jnp.where(lanes == k - 16, m, d1)
        i0 = jnp.where(lanes == k, idx, i0)
        i1 = jnp.where(lanes == k - 16, idx, i1)
        new_carry.append((d0, d1, i0, i1))
      return tuple(new_carry)

    init = tuple(
        (jnp.zeros((L,), jnp.float32), jnp.zeros((L,), jnp.float32),
         jnp.zeros((L,), jnp.int32), jnp.zeros((L,), jnp.int32))
        for _ in range(NQ))
    res = lax.fori_loop(0, K_NN, round_body, init)

    for p in range(NQ):
      d0, d1, i0, i1 = res[p]
      v0 = _sqrt16(jnp.maximum(d0, 1e-12))
      v1 = _sqrt16(jnp.maximum(d1, 1e-12))
      tot = jnp.sum(v0 + v1)
      od_ref[qi + p, pl.ds(0, L)] = v0 / tot
      od_ref[qi + p, pl.ds(L, L)] = v1 / tot
      oi_ref[qi + p, pl.ds(0, L)] = i0
      oi_ref[qi + p, pl.ds(L, L)] = i1

  pltpu.async_copy(od_ref, outd_hbm.at[c, s], sem).wait()
  pltpu.async_copy(oi_ref, outi_hbm.at[c, s], sem).wait()


@jax.jit
def kernel(src_pts, tgt_pts):
  src_xyz = src_pts[:, :3, :]
  tgt_xyz = tgt_pts[:, :3, :]

  mesh = plsc.VectorSubcoreMesh(core_axis_name="core", subcore_axis_name="subcore")
  cp = pltpu.CompilerParams(use_tc_tiling_on_sc=False)
  if "needs_layout_passes" in pltpu.CompilerParams.__dataclass_fields__:
    cp = dataclasses.replace(cp, needs_layout_passes=False)

  fn = pl.kernel(
      _sc_body,
      out_type=(
          jax.ShapeDtypeStruct((B, 16, QPT, K_NN), jnp.float32),
          jax.ShapeDtypeStruct((B, 16, QPT, K_NN), jnp.int32),
      ),
      mesh=mesh,
      scratch_types=[
          pltpu.VMEM((2, N), jnp.float32),
          pltpu.VMEM((N,), jnp.float32),
          pltpu.VMEM((3, QPT), jnp.float32),
          pltpu.VMEM((N,), jnp.float32),
          pltpu.VMEM((N,), jnp.float32),
          pltpu.VMEM((N,), jnp.float32),
          pltpu.VMEM((N,), jnp.float32),
          pltpu.VMEM((NGROUP, L), jnp.float32),
          pltpu.VMEM((NGROUP, L), jnp.float32),
          pltpu.VMEM((NGROUP, L), jnp.float32),
          pltpu.VMEM((NGROUP, L), jnp.float32),
          pltpu.VMEM((NSUPER, L), jnp.float32),
          pltpu.VMEM((NSUPER, L), jnp.float32),
          pltpu.VMEM((NSUPER, L), jnp.float32),
          pltpu.VMEM((NSUPER, L), jnp.float32),
          pltpu.VMEM((QPT, K_NN), jnp.float32),
          pltpu.VMEM((QPT, K_NN), jnp.int32),
          pltpu.SemaphoreType.DMA,
      ],
      compiler_params=cp,
  )
  outd, outi = fn(src_xyz, tgt_xyz)
  return outd.reshape(B, Q, K_NN), outi.reshape(B, Q, K_NN)
